# baseline (speedup 1.0000x reference)
import functools

import jax
import jax.numpy as jnp
from jax import lax
from jax.experimental import pallas as pl
from jax.experimental.pallas import tpu as pltpu
from jax.experimental.pallas import tpu_sc as plsc


def _make_gather(N, D, NC, NS):
    NW = NC * NS
    n_per_w = N // NW
    chunk = min(n_per_w, 512)
    n_chunks = n_per_w // chunk
    mesh = plsc.VectorSubcoreMesh(core_axis_name="c", subcore_axis_name="s")

    @functools.partial(
        pl.kernel,
        mesh=mesh,
        out_type=jax.ShapeDtypeStruct((N, 2 * D), jnp.float32),
        scratch_types=[
            pltpu.VMEM((n_per_w,), jnp.int32),
            pltpu.VMEM((chunk, 2 * D), jnp.float32),
            pltpu.SemaphoreType.DMA,
        ],
    )
    def gather_k(fused_hbm, idx_hbm, fused_out, idx_v, rows_v, sem):
        wid = lax.axis_index("s") * NC + lax.axis_index("c")
        base = wid * n_per_w
        pltpu.sync_copy(idx_hbm.at[pl.ds(base, n_per_w)], idx_v)
        for c in range(n_chunks):
            pltpu.async_copy(
                fused_hbm.at[idx_v.at[pl.ds(c * chunk, chunk)]], rows_v, sem
            ).wait()
            pltpu.sync_copy(
                rows_v, fused_out.at[pl.ds(base + c * chunk, chunk)]
            )

    return gather_k


def kernel(x, position_ids, cos_cached, sin_cached):
    B, S = position_ids.shape
    D = cos_cached.shape[1]
    N = B * S
    info = plsc.get_sparse_core_info()
    gather_k = _make_gather(N, D, info.num_cores, info.num_subcores)
    fused = jnp.concatenate([cos_cached, sin_cached], axis=1)
    idx = position_ids.reshape(N)
    fused_out = gather_k(fused, idx)
    return (fused_out[:, :D].reshape(B, S, D).astype(x.dtype),
            fused_out[:, D:].reshape(B, S, D).astype(x.dtype))
